# Initial kernel scaffold; baseline (speedup 1.0000x reference)
import jax
import jax.numpy as jnp
from jax.experimental import pallas as pl


def kernel(X0, X1_idx, X2_idx, L0_idx, L1u_idx, L1d_idx, L2_idx, B1_row, B1_col, B1_val, B2_row, B2_col, B2_val, Ws, bs, a1w, a1b, a2w, a2b, triW, trib, prelu_w):
    raise NotImplementedError("write your pallas kernel here")


# speedup vs baseline: 1.1971x; 1.1971x over previous
import jax
import jax.numpy as jnp
from jax.experimental import pallas as pl
from jax.experimental.pallas import tpu as pltpu

_R = 512


def _pad_rows(x, r=_R):
    n = x.shape[0]
    p = (-n) % r
    if p:
        x = jnp.pad(x, ((0, p),) + ((0, 0),) * (x.ndim - 1))
    return x


def _binarize_body(x_ref, o_ref):
    o_ref[...] = (x_ref[...] != 0).astype(jnp.float32)


def _binarize(x):
    n, d = x.shape
    xp = _pad_rows(x)
    g = xp.shape[0] // _R
    o = pl.pallas_call(
        _binarize_body,
        grid=(g,),
        in_specs=[pl.BlockSpec((_R, d), lambda i: (i, 0))],
        out_specs=pl.BlockSpec((_R, d), lambda i: (i, 0)),
        out_shape=jax.ShapeDtypeStruct(xp.shape, jnp.float32),
    )(xp)
    return o[:n]


def _satdense_body(x_ref, w_ref, b_ref, a1_ref, a2_ref, h_ref, s1_ref, s2_ref):
    x = x_ref[...]
    h = jnp.dot(x, w_ref[...], preferred_element_type=jnp.float32) + b_ref[...]
    h_ref[...] = h
    s1_ref[...] = jnp.sum(h * a1_ref[...], axis=1, keepdims=True)
    s2_ref[...] = jnp.sum(h * a2_ref[...], axis=1, keepdims=True)


def _satdense(x, W, b, aw1, aw2):
    n, d = x.shape
    xp = _pad_rows(x)
    npad = xp.shape[0]
    g = npad // _R
    h, s1, s2 = pl.pallas_call(
        _satdense_body,
        grid=(g,),
        in_specs=[
            pl.BlockSpec((_R, d), lambda i: (i, 0)),
            pl.BlockSpec((d, d), lambda i: (0, 0)),
            pl.BlockSpec((1, d), lambda i: (0, 0)),
            pl.BlockSpec((1, d), lambda i: (0, 0)),
            pl.BlockSpec((1, d), lambda i: (0, 0)),
        ],
        out_specs=[
            pl.BlockSpec((_R, d), lambda i: (i, 0)),
            pl.BlockSpec((_R, 1), lambda i: (i, 0)),
            pl.BlockSpec((_R, 1), lambda i: (i, 0)),
        ],
        out_shape=[
            jax.ShapeDtypeStruct((npad, d), jnp.float32),
            jax.ShapeDtypeStruct((npad, 1), jnp.float32),
            jax.ShapeDtypeStruct((npad, 1), jnp.float32),
        ],
    )(xp, W, b[None, :], aw1[None, :], aw2[None, :])
    return h[:n], s1[:n, 0], s2[:n, 0]


def _linear_body(x_ref, w_ref, b_ref, o_ref):
    o_ref[...] = (
        jnp.dot(x_ref[...], w_ref[...], preferred_element_type=jnp.float32)
        + b_ref[...]
    )


def _linear(x, W, b):
    n, d = x.shape
    xp = _pad_rows(x)
    g = xp.shape[0] // _R
    o = pl.pallas_call(
        _linear_body,
        grid=(g,),
        in_specs=[
            pl.BlockSpec((_R, d), lambda i: (i, 0)),
            pl.BlockSpec((d, d), lambda i: (0, 0)),
            pl.BlockSpec((1, d), lambda i: (0, 0)),
        ],
        out_specs=pl.BlockSpec((_R, d), lambda i: (i, 0)),
        out_shape=jax.ShapeDtypeStruct(xp.shape, jnp.float32),
    )(xp, W, b[None, :])
    return o[:n]


def _prelu_add_body(a_ref, b_ref, w_ref, o_ref):
    t = a_ref[...] + b_ref[...]
    w = w_ref[0, 0]
    o_ref[...] = jnp.maximum(t, 0.0) + w * jnp.minimum(t, 0.0)


def _prelu_add(a, b, prelu_w):
    n, d = a.shape
    ap = _pad_rows(a)
    bp = _pad_rows(b)
    g = ap.shape[0] // _R
    o = pl.pallas_call(
        _prelu_add_body,
        grid=(g,),
        in_specs=[
            pl.BlockSpec((_R, d), lambda i: (i, 0)),
            pl.BlockSpec((_R, d), lambda i: (i, 0)),
            pl.BlockSpec((1, 1), lambda i: (0, 0), memory_space=pltpu.SMEM),
        ],
        out_specs=pl.BlockSpec((_R, d), lambda i: (i, 0)),
        out_shape=jax.ShapeDtypeStruct(ap.shape, jnp.float32),
    )(ap, bp, prelu_w.reshape(1, 1))
    return o[:n]


def _avg3_body(a_ref, b_ref, c_ref, o_ref):
    o_ref[...] = (a_ref[...] + b_ref[...] + c_ref[...]) * (1.0 / 3.0)


def _avg3(a, b, c):
    n, d = a.shape
    ap, bp, cp = _pad_rows(a), _pad_rows(b), _pad_rows(c)
    g = ap.shape[0] // _R
    o = pl.pallas_call(
        _avg3_body,
        grid=(g,),
        in_specs=[
            pl.BlockSpec((_R, d), lambda i: (i, 0)),
            pl.BlockSpec((_R, d), lambda i: (i, 0)),
            pl.BlockSpec((_R, d), lambda i: (i, 0)),
        ],
        out_specs=pl.BlockSpec((_R, d), lambda i: (i, 0)),
        out_shape=jax.ShapeDtypeStruct(ap.shape, jnp.float32),
    )(ap, bp, cp)
    return o[:n]


def _sat_edges(h, s1, s2, r, c, n):
    m = jnp.max(s1) + jnp.max(s2)
    v = s1[r] + s2[c] - m
    e = jnp.exp(v)
    z = jax.ops.segment_sum(e, r, num_segments=n)
    attn = e / z[r]
    return jax.ops.segment_sum(attn[:, None] * h[c], r, num_segments=n)


def _sat(x, idx, W, b, aw1, ab1, aw2, ab2, n):
    h, s1, s2 = _satdense(x, W, b, aw1, aw2)
    return _sat_edges(h, s1 + ab1, s2 + ab2, idx[0], idx[1], n)


def _spmm(row, col, val, x, n):
    return jax.ops.segment_sum(val[:, None] * x[col], row, num_segments=n)


def kernel(X0, X1_idx, X2_idx, L0_idx, L1u_idx, L1d_idx, L2_idx, B1_row,
           B1_col, B1_val, B2_row, B2_col, B2_val, Ws, bs, a1w, a1b, a2w,
           a2b, triW, trib, prelu_w):
    n0 = X0.shape[0]
    n1 = X1_idx.shape[0]
    n2 = X2_idx.shape[0]
    X0b = _binarize(X0)
    X1 = X0b[X1_idx[:, 0]] * X0b[X1_idx[:, 1]]
    X2 = X0b[X2_idx[:, 0]] * X0b[X2_idx[:, 1]] * X0b[X2_idx[:, 2]]

    h0 = _prelu_add(
        _sat(X0b, L0_idx, Ws[0], bs[0], a1w[0], a1b[0], a2w[0], a2b[0], n0),
        _sat(X0b, L0_idx, Ws[1], bs[1], a1w[1], a1b[1], a2w[1], a2b[1], n0),
        prelu_w,
    )
    h1 = _prelu_add(
        _sat(X1, L1u_idx, Ws[2], bs[2], a1w[2], a1b[2], a2w[2], a2b[2], n1),
        _sat(X1, L1d_idx, Ws[3], bs[3], a1w[3], a1b[3], a2w[3], a2b[3], n1),
        prelu_w,
    )
    h2 = _prelu_add(
        _sat(X2, L2_idx, Ws[4], bs[4], a1w[4], a1b[4], a2w[4], a2b[4], n2),
        _sat(X2, L2_idx, Ws[5], bs[5], a1w[5], a1b[5], a2w[5], a2b[5], n2),
        prelu_w,
    )
    tri = _linear(_spmm(B2_row, B2_col, B2_val, h2, n1), triW, trib)
    return _avg3(h0, _spmm(B1_row, B1_col, B1_val, h1, n0),
                 _spmm(B1_row, B1_col, B1_val, tri, n0))


# speedup vs baseline: 6.1608x; 5.1464x over previous
import functools

import jax
import jax.numpy as jnp
from jax import lax
from jax.experimental import pallas as pl
from jax.experimental.pallas import tpu as pltpu
from jax.experimental.pallas import tpu_sc as plsc

_R = 512
_EB = 1024
_CH = _EB // 128


def _edge_exp_z_sc(s1f, s2f, r, c, n):
    nnz = r.shape[0]
    info = plsc.get_sparse_core_info()
    NC, NS = int(info.num_cores), int(info.num_subcores)
    NW = NC * NS
    grain = NW * _EB
    nnz_p = -(-nnz // grain) * grain
    nzp = -(-(n + 1) // (NS * 128)) * (NS * 128)
    pwr = (nnz_p // NW) // 128
    chunks = pwr // _CH
    zsl = nzp // NS
    zrows = zsl // 128

    r_p = jnp.concatenate(
        [r.astype(jnp.int32), jnp.full((nnz_p - nnz,), n, jnp.int32)]
    ).reshape(nnz_p // 128, 128)
    c_p = jnp.concatenate(
        [c.astype(jnp.int32), jnp.zeros((nnz_p - nnz,), jnp.int32)]
    ).reshape(nnz_p // 128, 128)
    s1_p = jnp.concatenate([s1f, jnp.full((nzp - n,), -1e30, jnp.float32)])
    s2_p = jnp.concatenate([s2f, jnp.full((nzp - n,), -1e30, jnp.float32)])

    mesh = plsc.VectorSubcoreMesh(core_axis_name="c", subcore_axis_name="s")

    @functools.partial(
        pl.kernel,
        mesh=mesh,
        out_type=[
            jax.ShapeDtypeStruct((nnz_p // 128, 128), jnp.float32),
            jax.ShapeDtypeStruct((NC * nzp,), jnp.float32),
        ],
        scratch_types=[
            pltpu.VMEM((_CH, 128), jnp.int32),
            pltpu.VMEM((_CH, 128), jnp.int32),
            pltpu.VMEM((_CH, 128), jnp.float32),
            pltpu.VMEM((_CH, 128), jnp.float32),
            pltpu.VMEM((_CH, 128), jnp.float32),
            pltpu.VMEM((128,), jnp.float32),
            pltpu.SemaphoreType.DMA,
            pltpu.VMEM_SHARED((nzp,), jnp.float32),
        ],
    )
    def k(s1_hbm, s2_hbm, r_hbm, c_hbm, e_hbm, zp_hbm,
          rv, cv, g1, g2, ev, zrow, sem, zsh):
        cid = lax.axis_index("c")
        sid = lax.axis_index("s")
        wid = sid * NC + cid
        for kk in range(8):
            zrow[pl.ds(kk * 16, 16)] = jnp.zeros((16,), jnp.float32)
        zoff = sid * zsl

        def zinit(j, carry):
            pltpu.sync_copy(zrow, zsh.at[pl.ds(zoff + j * 128, 128)])
            return carry

        lax.fori_loop(0, zrows, zinit, 0)
        plsc.subcore_barrier()

        rbase = wid * pwr

        def chunk(j, carry):
            rb = rbase + j * _CH
            pltpu.sync_copy(r_hbm.at[pl.ds(rb, _CH)], rv)
            pltpu.sync_copy(c_hbm.at[pl.ds(rb, _CH)], cv)
            cps = []
            for t in range(_CH):
                cps.append(pltpu.async_copy(s1_hbm.at[rv.at[t]], g1.at[t], sem))
                cps.append(pltpu.async_copy(s2_hbm.at[cv.at[t]], g2.at[t], sem))
            for cp in cps:
                cp.wait()
            for t in range(_CH):
                for kk in range(8):
                    sl = pl.ds(kk * 16, 16)
                    ev[t, sl] = jnp.exp(g1[t, sl] + g2[t, sl])
            pltpu.sync_copy(ev, e_hbm.at[pl.ds(rb, _CH)])
            for t in range(_CH):
                pltpu.sync_copy(ev.at[t], zsh.at[rv.at[t]], add=True)
            return carry

        lax.fori_loop(0, chunks, chunk, 0)
        plsc.subcore_barrier()

        zo = cid * nzp + zoff

        def zout(j, carry):
            pltpu.sync_copy(zsh.at[pl.ds(zoff + j * 128, 128)], zrow)
            pltpu.sync_copy(zrow, zp_hbm.at[pl.ds(zo + j * 128, 128)])
            return carry

        lax.fori_loop(0, zrows, zout, 0)

    e2, zp = k(s1_p, s2_p, r_p, c_p)
    e = e2.reshape(-1)[:nnz]
    z = zp.reshape(NC, nzp).sum(0)[:n]
    return e, z


def _pad_rows(x, r=_R):
    n = x.shape[0]
    p = (-n) % r
    if p:
        x = jnp.pad(x, ((0, p),) + ((0, 0),) * (x.ndim - 1))
    return x


def _binarize_body(x_ref, o_ref):
    o_ref[...] = (x_ref[...] != 0).astype(jnp.float32)


def _binarize(x):
    n, d = x.shape
    xp = _pad_rows(x)
    g = xp.shape[0] // _R
    o = pl.pallas_call(
        _binarize_body,
        grid=(g,),
        in_specs=[pl.BlockSpec((_R, d), lambda i: (i, 0))],
        out_specs=pl.BlockSpec((_R, d), lambda i: (i, 0)),
        out_shape=jax.ShapeDtypeStruct(xp.shape, jnp.float32),
    )(xp)
    return o[:n]


def _satdense_body(x_ref, w_ref, b_ref, a1_ref, a2_ref, h_ref, s1_ref, s2_ref):
    x = x_ref[...]
    h = jnp.dot(x, w_ref[...], preferred_element_type=jnp.float32) + b_ref[...]
    h_ref[...] = h
    s1_ref[...] = jnp.sum(h * a1_ref[...], axis=1, keepdims=True)
    s2_ref[...] = jnp.sum(h * a2_ref[...], axis=1, keepdims=True)


def _satdense(x, W, b, aw1, aw2):
    n, d = x.shape
    xp = _pad_rows(x)
    npad = xp.shape[0]
    g = npad // _R
    h, s1, s2 = pl.pallas_call(
        _satdense_body,
        grid=(g,),
        in_specs=[
            pl.BlockSpec((_R, d), lambda i: (i, 0)),
            pl.BlockSpec((d, d), lambda i: (0, 0)),
            pl.BlockSpec((1, d), lambda i: (0, 0)),
            pl.BlockSpec((1, d), lambda i: (0, 0)),
            pl.BlockSpec((1, d), lambda i: (0, 0)),
        ],
        out_specs=[
            pl.BlockSpec((_R, d), lambda i: (i, 0)),
            pl.BlockSpec((_R, 1), lambda i: (i, 0)),
            pl.BlockSpec((_R, 1), lambda i: (i, 0)),
        ],
        out_shape=[
            jax.ShapeDtypeStruct((npad, d), jnp.float32),
            jax.ShapeDtypeStruct((npad, 1), jnp.float32),
            jax.ShapeDtypeStruct((npad, 1), jnp.float32),
        ],
    )(xp, W, b[None, :], aw1[None, :], aw2[None, :])
    return h[:n], s1[:n, 0], s2[:n, 0]


def _linear_body(x_ref, w_ref, b_ref, o_ref):
    o_ref[...] = (
        jnp.dot(x_ref[...], w_ref[...], preferred_element_type=jnp.float32)
        + b_ref[...]
    )


def _linear(x, W, b):
    n, d = x.shape
    xp = _pad_rows(x)
    g = xp.shape[0] // _R
    o = pl.pallas_call(
        _linear_body,
        grid=(g,),
        in_specs=[
            pl.BlockSpec((_R, d), lambda i: (i, 0)),
            pl.BlockSpec((d, d), lambda i: (0, 0)),
            pl.BlockSpec((1, d), lambda i: (0, 0)),
        ],
        out_specs=pl.BlockSpec((_R, d), lambda i: (i, 0)),
        out_shape=jax.ShapeDtypeStruct(xp.shape, jnp.float32),
    )(xp, W, b[None, :])
    return o[:n]


def _prelu_add_body(a_ref, b_ref, w_ref, o_ref):
    t = a_ref[...] + b_ref[...]
    w = w_ref[0, 0]
    o_ref[...] = jnp.maximum(t, 0.0) + w * jnp.minimum(t, 0.0)


def _prelu_add(a, b, prelu_w):
    n, d = a.shape
    ap = _pad_rows(a)
    bp = _pad_rows(b)
    g = ap.shape[0] // _R
    o = pl.pallas_call(
        _prelu_add_body,
        grid=(g,),
        in_specs=[
            pl.BlockSpec((_R, d), lambda i: (i, 0)),
            pl.BlockSpec((_R, d), lambda i: (i, 0)),
            pl.BlockSpec((1, 1), lambda i: (0, 0), memory_space=pltpu.SMEM),
        ],
        out_specs=pl.BlockSpec((_R, d), lambda i: (i, 0)),
        out_shape=jax.ShapeDtypeStruct(ap.shape, jnp.float32),
    )(ap, bp, prelu_w.reshape(1, 1))
    return o[:n]


def _avg3_body(a_ref, b_ref, c_ref, o_ref):
    o_ref[...] = (a_ref[...] + b_ref[...] + c_ref[...]) * (1.0 / 3.0)


def _avg3(a, b, c):
    n, d = a.shape
    ap, bp, cp = _pad_rows(a), _pad_rows(b), _pad_rows(c)
    g = ap.shape[0] // _R
    o = pl.pallas_call(
        _avg3_body,
        grid=(g,),
        in_specs=[
            pl.BlockSpec((_R, d), lambda i: (i, 0)),
            pl.BlockSpec((_R, d), lambda i: (i, 0)),
            pl.BlockSpec((_R, d), lambda i: (i, 0)),
        ],
        out_specs=pl.BlockSpec((_R, d), lambda i: (i, 0)),
        out_shape=jax.ShapeDtypeStruct(ap.shape, jnp.float32),
    )(ap, bp, cp)
    return o[:n]


def _sat(x, idx, W, b, aw1, ab1, aw2, ab2, n):
    h, s1, s2 = _satdense(x, W, b, aw1, aw2)
    m = jnp.max(s1) + jnp.max(s2)
    r = idx[0]
    c = idx[1]
    e, z = _edge_exp_z_sc(s1 + (ab1 - m), s2 + ab2, r, c, n)
    rows = jax.ops.segment_sum(e[:, None] * h[c], r, num_segments=n)
    zs = jnp.where(z > 0, z, 1.0)
    return rows / zs[:, None]


def _spmm(row, col, val, x, n):
    return jax.ops.segment_sum(val[:, None] * x[col], row, num_segments=n)


def kernel(X0, X1_idx, X2_idx, L0_idx, L1u_idx, L1d_idx, L2_idx, B1_row,
           B1_col, B1_val, B2_row, B2_col, B2_val, Ws, bs, a1w, a1b, a2w,
           a2b, triW, trib, prelu_w):
    n0 = X0.shape[0]
    n1 = X1_idx.shape[0]
    n2 = X2_idx.shape[0]
    X0b = _binarize(X0)
    X1 = X0b[X1_idx[:, 0]] * X0b[X1_idx[:, 1]]
    X2 = X0b[X2_idx[:, 0]] * X0b[X2_idx[:, 1]] * X0b[X2_idx[:, 2]]

    h0 = _prelu_add(
        _sat(X0b, L0_idx, Ws[0], bs[0], a1w[0], a1b[0], a2w[0], a2b[0], n0),
        _sat(X0b, L0_idx, Ws[1], bs[1], a1w[1], a1b[1], a2w[1], a2b[1], n0),
        prelu_w,
    )
    h1 = _prelu_add(
        _sat(X1, L1u_idx, Ws[2], bs[2], a1w[2], a1b[2], a2w[2], a2b[2], n1),
        _sat(X1, L1d_idx, Ws[3], bs[3], a1w[3], a1b[3], a2w[3], a2b[3], n1),
        prelu_w,
    )
    h2 = _prelu_add(
        _sat(X2, L2_idx, Ws[4], bs[4], a1w[4], a1b[4], a2w[4], a2b[4], n2),
        _sat(X2, L2_idx, Ws[5], bs[5], a1w[5], a1b[5], a2w[5], a2b[5], n2),
        prelu_w,
    )
    tri = _linear(_spmm(B2_row, B2_col, B2_val, h2, n1), triW, trib)
    both = _spmm(B1_row, B1_col, B1_val,
                 jnp.concatenate([h1, tri], axis=1), n0)
    d = h0.shape[1]
    return _avg3(h0, both[:, :d], both[:, d:])
